# baseline (speedup 1.0000x reference)
import functools

import jax
import jax.numpy as jnp
from jax.experimental import pallas as pl
from jax.experimental.pallas import tpu as pltpu


def _softmax_linear_t_block(x_ref, wt_ref, b_ref, ot_ref):
    logits_t = jax.lax.dot_general(
        wt_ref[...], x_ref[...],
        dimension_numbers=(((1,), (1,)), ((), ())),
        preferred_element_type=jnp.float32)
    logits_t = logits_t + jnp.transpose(b_ref[...])
    m = jnp.max(logits_t, axis=0, keepdims=True)
    e = jnp.exp(logits_t - m)
    s = jnp.sum(e, axis=0, keepdims=True)
    ot_ref[...] = e * pl.reciprocal(s, approx=False)


@jax.jit
def kernel(x, w_fused, b_fused):
    B, D = x.shape
    C = w_fused.shape[1]

    block_b = min(4096, B)
    n_blocks = pl.cdiv(B, block_b)
    n_outer = 2 if n_blocks % 2 == 0 else 1
    n_inner = n_blocks // n_outer

    w_t = jnp.transpose(w_fused.astype(jnp.float32))
    b_2d = jnp.reshape(b_fused.astype(jnp.float32), (1, C))

    out_t = pl.pallas_call(
        _softmax_linear_t_block,
        out_shape=jax.ShapeDtypeStruct((C, B), jnp.float32),
        grid=(n_outer, n_inner),
        in_specs=[
            pl.BlockSpec((block_b, D), lambda i, j: (i * n_inner + j, 0)),
            pl.BlockSpec((C, D), lambda i, j: (0, 0)),
            pl.BlockSpec((1, C), lambda i, j: (0, 0)),
        ],
        out_specs=pl.BlockSpec((C, block_b), lambda i, j: (0, i * n_inner + j)),
        compiler_params=pltpu.CompilerParams(
            dimension_semantics=("parallel", "arbitrary")),
    )(x, w_t, b_2d)
    return jnp.transpose(out_t)


# speedup vs baseline: 1.3723x; 1.3723x over previous
import functools

import jax
import jax.numpy as jnp
from jax.experimental import pallas as pl
from jax.experimental.pallas import tpu as pltpu


def _softmax_linear_t_block(x_ref, wt_ref, b_ref, ot_ref):
    logits_t = jax.lax.dot_general(
        wt_ref[...], x_ref[...],
        dimension_numbers=(((1,), (1,)), ((), ())),
        preferred_element_type=jnp.float32)
    logits_t = logits_t + jnp.transpose(b_ref[...])
    m = jnp.max(logits_t, axis=0, keepdims=True)
    e = jnp.exp(logits_t - m)
    s = jnp.sum(e, axis=0, keepdims=True)
    ot_ref[...] = e * pl.reciprocal(s, approx=False)


@jax.jit
def kernel(x, w_fused, b_fused):
    B, D = x.shape
    C = w_fused.shape[1]

    block_b = min(16384, B)
    n_blocks = pl.cdiv(B, block_b)

    w_t = jnp.transpose(w_fused.astype(jnp.float32))
    b_2d = jnp.reshape(b_fused.astype(jnp.float32), (1, C))

    out_t = pl.pallas_call(
        _softmax_linear_t_block,
        out_shape=jax.ShapeDtypeStruct((C, B), jnp.float32),
        grid=(n_blocks,),
        in_specs=[
            pl.BlockSpec((block_b, D), lambda i: (i, 0)),
            pl.BlockSpec((C, D), lambda i: (0, 0)),
            pl.BlockSpec((1, C), lambda i: (0, 0)),
        ],
        out_specs=pl.BlockSpec((C, block_b), lambda i: (0, i)),
        compiler_params=pltpu.CompilerParams(
            dimension_semantics=("parallel",)),
    )(x, w_t, b_2d)
    return jnp.transpose(out_t)
